# baseline (speedup 1.0000x reference)
import functools

import jax
import jax.numpy as jnp
from jax import lax
from jax.experimental import pallas as pl
from jax.experimental.pallas import tpu as pltpu
from jax.experimental.pallas import tpu_sc as plsc

_D = 1024
_MAX_LEN = 32768
_SEQ = 8192
_NC = 2
_NS = 16
_NW = _NC * _NS
_B_PER_W = _SEQ // _NW
_C = 16
_NCHUNK = _B_PER_W // _C
_NBUF = 6

_mesh = plsc.VectorSubcoreMesh(core_axis_name="c", subcore_axis_name="s")


@functools.partial(
    pl.kernel,
    mesh=_mesh,
    out_type=jax.ShapeDtypeStruct((_SEQ, _D), jnp.float32),
    scratch_types=(
        [pltpu.VMEM((_NCHUNK, _C), jnp.int32),
         pltpu.VMEM((_NBUF, _C, _D), jnp.float32)]
        + [pltpu.SemaphoreType.DMA] * (2 * _NBUF)
    ),
)
def _gather(table_hbm, idx_hbm, out_hbm, idx_v, bufs, *sems):
    wid = lax.axis_index("s") * _NC + lax.axis_index("c")
    base = wid * _B_PER_W
    gsem = sems[:_NBUF]
    ssem = sems[_NBUF:]
    pltpu.sync_copy(idx_hbm.at[wid], idx_v)

    def start_gather(c):
        b = c % _NBUF
        return pltpu.async_copy(table_hbm.at[idx_v.at[c]], bufs.at[b], gsem[b])

    def start_store(c):
        b = c % _NBUF
        return pltpu.async_copy(
            bufs.at[b], out_hbm.at[pl.ds(base + c * _C, _C)], ssem[b]
        )

    gathers = [None] * _NCHUNK
    stores = [None] * _NCHUNK
    for c in range(min(_NBUF - 1, _NCHUNK)):
        gathers[c] = start_gather(c)
    for c in range(_NCHUNK):
        n = c + _NBUF - 1
        if n < _NCHUNK:
            if c >= 1:
                stores[c - 1].wait()
            gathers[n] = start_gather(n)
        gathers[c].wait()
        stores[c] = start_store(c)
    for c in range(max(0, _NCHUNK - _NBUF + 1), _NCHUNK):
        stores[c].wait()
    if _NCHUNK >= _NBUF:
        stores[_NCHUNK - _NBUF].wait()


def kernel(position_ids, PosEnc):
    table = PosEnc.reshape(_MAX_LEN, _D)
    idx = position_ids.astype(jnp.int32).reshape(_NW, _NCHUNK, _C)
    out = _gather(table, idx)
    return out.reshape(1, _SEQ, _D)


# speedup vs baseline: 1.0087x; 1.0087x over previous
import functools

import jax
import jax.numpy as jnp
from jax import lax
from jax.experimental import pallas as pl
from jax.experimental.pallas import tpu as pltpu
from jax.experimental.pallas import tpu_sc as plsc

_D = 1024
_MAX_LEN = 32768
_SEQ = 8192
_NC = 2
_NS = 16
_NW = _NC * _NS
_B_PER_W = _SEQ // _NW
_C = 16
_NCHUNK = _B_PER_W // _C
_NBUF = 6

_mesh = plsc.VectorSubcoreMesh(core_axis_name="c", subcore_axis_name="s")


@functools.partial(
    pl.kernel,
    mesh=_mesh,
    out_type=jax.ShapeDtypeStruct((_SEQ, _D), jnp.float32),
    scratch_types=(
        [pltpu.VMEM((_NCHUNK, _C), jnp.int32),
         pltpu.VMEM((_NBUF, _C, _D), jnp.float32)]
        + [pltpu.SemaphoreType.DMA] * (2 * _NBUF)
    ),
)
def _gather(table_hbm, idx_hbm, out_hbm, idx_v, bufs, *sems):
    wid = lax.axis_index("s") * _NC + lax.axis_index("c")
    base = wid * _B_PER_W
    gsem = sems[:_NBUF]
    ssem = sems[_NBUF:]
    pltpu.sync_copy(idx_hbm.at[wid], idx_v)

    def start_gather(c):
        b = c % _NBUF
        return pltpu.async_copy(table_hbm.at[idx_v.at[c]], bufs.at[b], gsem[b])

    def start_store(c):
        b = c % _NBUF
        return pltpu.async_copy(
            bufs.at[b], out_hbm.at[pl.ds(base + c * _C, _C)], ssem[b]
        )

    HB = _NBUF // 2
    gathers = [None] * _NCHUNK
    stores = [None] * _NCHUNK

    def ring_chunks(r):
        return list(range(r, _NCHUNK, 2))

    primed = {0: 0, 1: 0}
    order = []
    for i in range(HB - 1):
        for r in (0, 1):
            ch = ring_chunks(r)
            if i < len(ch):
                gathers[ch[i]] = start_gather(ch[i])
                primed[r] += 1
    for step in range(_NCHUNK // 2):
        for r in (0, 1):
            ch = ring_chunks(r)
            c = ch[step]
            n_i = step + HB - 1
            if n_i < len(ch):
                if step >= 1:
                    stores[ch[step - 1]].wait()
                gathers[ch[n_i]] = start_gather(ch[n_i])
            gathers[c].wait()
            stores[c] = start_store(c)
    for r in (0, 1):
        ch = ring_chunks(r)
        for c in ch[max(0, len(ch) - HB):]:
            stores[c].wait()


def kernel(position_ids, PosEnc):
    table = PosEnc.reshape(_MAX_LEN, _D)
    idx = position_ids.astype(jnp.int32).reshape(_NW, _NCHUNK, _C)
    out = _gather(table, idx)
    return out.reshape(1, _SEQ, _D)
